# Initial kernel scaffold; baseline (speedup 1.0000x reference)
import jax
import jax.numpy as jnp
from jax.experimental import pallas as pl


def kernel(x, edge_index, W_enc1, b_enc1, W_enc3, b_enc3, W_dec1, b_dec1, W_dec3, b_dec3):
    raise NotImplementedError("write your pallas kernel here")


# speedup vs baseline: 5.9098x; 5.9098x over previous
import functools

import jax
import jax.numpy as jnp
from jax import lax
from jax.experimental import pallas as pl
from jax.experimental.pallas import tpu as pltpu
from jax.experimental.pallas import tpu_sc as plsc

N_NODES = 10000
D_FEAT = 128
AUGD = 144
ROWS = 10112
N_EDGES = 320000
NC, NS = 2, 16
NW = NC * NS
K = 64
NCHUNK = 158
E_T = K * NCHUNK
NEP = NW * E_T
STRIPE = ROWS // NS

IN_DIM = 2 * D_FEAT
H2 = 192
EMB = 128


@functools.cache
def _make_sc_agg():
    mesh = plsc.VectorSubcoreMesh(
        core_axis_name="c", subcore_axis_name="s",
        num_cores=NC, num_subcores=NS)

    @functools.partial(
        pl.kernel,
        out_type=jax.ShapeDtypeStruct((NC, ROWS, AUGD), jnp.float32),
        mesh=mesh,
        scratch_types=[
            pltpu.VMEM((NCHUNK, K), jnp.int32),
            pltpu.VMEM((NCHUNK, K), jnp.int32),
            pltpu.VMEM((K, AUGD), jnp.float32),
            pltpu.VMEM((K, AUGD), jnp.float32),
            pltpu.VMEM_SHARED((ROWS, AUGD), jnp.float32),
            pltpu.SemaphoreType.DMA,
            pltpu.SemaphoreType.DMA,
        ],
        compiler_params=pltpu.CompilerParams(use_tc_tiling_on_sc=False),
    )
    def sc_agg(xaug_hbm, src_hbm, dst_hbm, parts_out,
               sidx, didx, buf0, buf1, acc, sem0, sem1):
        c = lax.axis_index("c")
        s = lax.axis_index("s")
        wid = c * NS + s

        pltpu.sync_copy(src_hbm.at[pl.ds(wid * NCHUNK, NCHUNK)], sidx)
        pltpu.sync_copy(dst_hbm.at[pl.ds(wid * NCHUNK, NCHUNK)], didx)

        def _zrow(i, _):
            for g in range(AUGD // 16):
                buf0[i, pl.ds(g * 16, 16)] = jnp.zeros((16,), jnp.float32)
            return _
        lax.fori_loop(0, K, _zrow, None)
        for kk in range(STRIPE // K):
            pltpu.sync_copy(buf0, acc.at[pl.ds(s * STRIPE + kk * K, K)])
        rem = STRIPE % K
        if rem:
            pltpu.sync_copy(buf0.at[pl.ds(0, rem)],
                            acc.at[pl.ds(s * STRIPE + (STRIPE // K) * K, rem)])
        plsc.subcore_barrier()

        pltpu.async_copy(xaug_hbm.at[sidx.at[0]], buf0, sem0)

        def body(i, _):
            j = 2 * i
            pltpu.async_copy(xaug_hbm.at[sidx.at[j + 1]], buf1, sem1)
            pltpu.make_async_copy(xaug_hbm.at[sidx.at[j]], buf0, sem0).wait()
            pltpu.sync_copy(buf0, acc.at[didx.at[j]], add=True)

            @pl.when(j + 2 < NCHUNK)
            def _():
                pltpu.async_copy(xaug_hbm.at[sidx.at[j + 2]], buf0, sem0)

            pltpu.make_async_copy(xaug_hbm.at[sidx.at[j + 1]], buf1, sem1).wait()
            pltpu.sync_copy(buf1, acc.at[didx.at[j + 1]], add=True)
            return _

        lax.fori_loop(0, NCHUNK // 2, body, None)

        plsc.subcore_barrier()
        pltpu.sync_copy(acc.at[pl.ds(s * STRIPE, STRIPE)],
                        parts_out.at[c, pl.ds(s * STRIPE, STRIPE)])

    return sc_agg


def _tc_dense_body(x_ref, parts_ref, w1_ref, b1_ref, w2_ref, b2_ref,
                   w3_ref, b3_ref, w4_ref, b4_ref, enc_ref, dec_ref):
    xs = x_ref[...]
    p = parts_ref[0] + parts_ref[1]
    cnt = p[:, D_FEAT:D_FEAT + 1]
    agg = p[:, :D_FEAT] / jnp.maximum(cnt, 1.0)
    col = lax.broadcasted_iota(jnp.int32, xs.shape, 1)
    xz = jnp.where(col == 0, 0.0, xs)
    aggz = jnp.where(col == 0, 0.0, agg)
    w1 = w1_ref[...]
    h = jnp.maximum(
        jnp.dot(xz, w1[:D_FEAT], preferred_element_type=jnp.float32)
        + jnp.dot(aggz, w1[D_FEAT:], preferred_element_type=jnp.float32)
        + b1_ref[...], 0.0)
    enc = jnp.dot(h, w2_ref[...], preferred_element_type=jnp.float32) + b2_ref[...]
    enc_ref[...] = enc
    h2 = jnp.maximum(
        jnp.dot(enc, w3_ref[...], preferred_element_type=jnp.float32)
        + b3_ref[...], 0.0)
    dec_ref[...] = (jnp.dot(h2, w4_ref[...], preferred_element_type=jnp.float32)
                    + b4_ref[...])


_TC_R = 1264


def _tc_dense(xp, parts, W_enc1, b_enc1, W_enc3, b_enc3,
              W_dec1, b_dec1, W_dec3, b_dec3):
    grid = (ROWS // _TC_R,)
    fixed = lambda i: (0, 0)
    enc, dec = pl.pallas_call(
        _tc_dense_body,
        grid=grid,
        in_specs=[
            pl.BlockSpec((_TC_R, D_FEAT), lambda i: (i, 0)),
            pl.BlockSpec((NC, _TC_R, AUGD), lambda i: (0, i, 0)),
            pl.BlockSpec((IN_DIM, H2), fixed),
            pl.BlockSpec((1, H2), fixed),
            pl.BlockSpec((H2, EMB), fixed),
            pl.BlockSpec((1, EMB), fixed),
            pl.BlockSpec((EMB, H2), fixed),
            pl.BlockSpec((1, H2), fixed),
            pl.BlockSpec((H2, IN_DIM), fixed),
            pl.BlockSpec((1, IN_DIM), fixed),
        ],
        out_specs=[
            pl.BlockSpec((_TC_R, EMB), lambda i: (i, 0)),
            pl.BlockSpec((_TC_R, IN_DIM), lambda i: (i, 0)),
        ],
        out_shape=[
            jax.ShapeDtypeStruct((ROWS, EMB), jnp.float32),
            jax.ShapeDtypeStruct((ROWS, IN_DIM), jnp.float32),
        ],
    )(xp, parts, W_enc1, b_enc1.reshape(1, H2), W_enc3, b_enc3.reshape(1, EMB),
      W_dec1, b_dec1.reshape(1, H2), W_dec3, b_dec3.reshape(1, IN_DIM))
    return enc, dec


def kernel(x, edge_index, W_enc1, b_enc1, W_enc3, b_enc3,
           W_dec1, b_dec1, W_dec3, b_dec3):
    xaug = jnp.concatenate(
        [x, jnp.ones((N_NODES, 1), jnp.float32),
         jnp.zeros((N_NODES, AUGD - D_FEAT - 1), jnp.float32)], axis=1)
    src = edge_index[0]
    dst = edge_index[1]
    pad = NEP - N_EDGES
    srcp = jnp.concatenate([src, jnp.zeros((pad,), jnp.int32)]).reshape(NW * NCHUNK, K)
    dstp = jnp.concatenate([dst, jnp.full((pad,), N_NODES, jnp.int32)]).reshape(NW * NCHUNK, K)

    parts = _make_sc_agg()(xaug, srcp, dstp)

    xp = jnp.pad(x, ((0, ROWS - N_NODES), (0, 0)))
    enc, dec = _tc_dense(xp, parts, W_enc1, b_enc1, W_enc3, b_enc3,
                         W_dec1, b_dec1, W_dec3, b_dec3)
    return enc[:N_NODES], dec[:N_NODES]


# speedup vs baseline: 6.4715x; 1.0950x over previous
import functools

import jax
import jax.numpy as jnp
from jax import lax
from jax.experimental import pallas as pl
from jax.experimental.pallas import tpu as pltpu
from jax.experimental.pallas import tpu_sc as plsc

N_NODES = 10000
D_FEAT = 128
AUGD = 144
ROWS = 10112
N_EDGES = 320000
NC, NS = 2, 16
NW = NC * NS
K = 64
NCHUNK = 158
E_T = K * NCHUNK
NEP = NW * E_T
STRIPE = ROWS // NS

IN_DIM = 2 * D_FEAT
H2 = 192
EMB = 128


@functools.cache
def _make_sc_agg():
    mesh = plsc.VectorSubcoreMesh(
        core_axis_name="c", subcore_axis_name="s",
        num_cores=NC, num_subcores=NS)

    @functools.partial(
        pl.kernel,
        out_type=jax.ShapeDtypeStruct((NC, ROWS, AUGD), jnp.float32),
        mesh=mesh,
        scratch_types=[
            pltpu.VMEM((NCHUNK, K), jnp.int32),
            pltpu.VMEM((NCHUNK, K), jnp.int32),
            pltpu.VMEM((K, AUGD), jnp.float32),
            pltpu.VMEM((K, AUGD), jnp.float32),
            pltpu.VMEM_SHARED((ROWS, AUGD), jnp.float32),
            pltpu.SemaphoreType.DMA,
            pltpu.SemaphoreType.DMA,
        ],
        compiler_params=pltpu.CompilerParams(use_tc_tiling_on_sc=False),
    )
    def sc_agg(xaug_hbm, src_hbm, dst_hbm, parts_out,
               sidx, didx, buf0, buf1, acc, sem0, sem1):
        c = lax.axis_index("c")
        s = lax.axis_index("s")
        wid = c * NS + s

        pltpu.sync_copy(src_hbm.at[pl.ds(wid * NCHUNK, NCHUNK)], sidx)
        pltpu.sync_copy(dst_hbm.at[pl.ds(wid * NCHUNK, NCHUNK)], didx)

        def _zrow(i, _):
            for g in range(AUGD // 16):
                buf0[i, pl.ds(g * 16, 16)] = jnp.zeros((16,), jnp.float32)
            return _
        lax.fori_loop(0, K, _zrow, None)
        for kk in range(STRIPE // K):
            pltpu.sync_copy(buf0, acc.at[pl.ds(s * STRIPE + kk * K, K)])
        rem = STRIPE % K
        if rem:
            pltpu.sync_copy(buf0.at[pl.ds(0, rem)],
                            acc.at[pl.ds(s * STRIPE + (STRIPE // K) * K, rem)])
        plsc.subcore_barrier()

        pltpu.async_copy(xaug_hbm.at[sidx.at[0]], buf0, sem0)

        def body(i, _):
            j = 2 * i
            pltpu.async_copy(xaug_hbm.at[sidx.at[j + 1]], buf1, sem1)
            pltpu.make_async_copy(xaug_hbm.at[sidx.at[j]], buf0, sem0).wait()
            pltpu.sync_copy(buf0, acc.at[didx.at[j]], add=True)

            @pl.when(j + 2 < NCHUNK)
            def _():
                pltpu.async_copy(xaug_hbm.at[sidx.at[j + 2]], buf0, sem0)

            pltpu.make_async_copy(xaug_hbm.at[sidx.at[j + 1]], buf1, sem1).wait()
            pltpu.sync_copy(buf1, acc.at[didx.at[j + 1]], add=True)
            return _

        lax.fori_loop(0, NCHUNK // 2, body, None)

        plsc.subcore_barrier()
        pltpu.sync_copy(acc.at[pl.ds(s * STRIPE, STRIPE)],
                        parts_out.at[c, pl.ds(s * STRIPE, STRIPE)])

    return sc_agg


def _tc_dense_body(x_ref, parts_ref, w1_ref, b1_ref, w2_ref, b2_ref,
                   w3_ref, b3_ref, w4_ref, b4_ref, enc_ref, dec_ref):
    xs = x_ref[...]
    p = parts_ref[0] + parts_ref[1]
    cnt = p[:, D_FEAT:D_FEAT + 1]
    agg = p[:, :D_FEAT] / jnp.maximum(cnt, 1.0)
    col = lax.broadcasted_iota(jnp.int32, xs.shape, 1)
    xz = jnp.where(col == 0, 0.0, xs)
    aggz = jnp.where(col == 0, 0.0, agg)
    w1 = w1_ref[...]
    h = jnp.maximum(
        jnp.dot(xz, w1[:D_FEAT], preferred_element_type=jnp.float32)
        + jnp.dot(aggz, w1[D_FEAT:], preferred_element_type=jnp.float32)
        + b1_ref[...], 0.0)
    enc = jnp.dot(h, w2_ref[...], preferred_element_type=jnp.float32) + b2_ref[...]
    enc_ref[...] = enc
    h2 = jnp.maximum(
        jnp.dot(enc, w3_ref[...], preferred_element_type=jnp.float32)
        + b3_ref[...], 0.0)
    dec_ref[...] = (jnp.dot(h2, w4_ref[...], preferred_element_type=jnp.float32)
                    + b4_ref[...])


_TC_R = 1008


def _tc_dense(xp, parts, W_enc1, b_enc1, W_enc3, b_enc3,
              W_dec1, b_dec1, W_dec3, b_dec3):
    grid = (-(-N_NODES // _TC_R),)
    fixed = lambda i: (0, 0)
    enc, dec = pl.pallas_call(
        _tc_dense_body,
        grid=grid,
        in_specs=[
            pl.BlockSpec((_TC_R, D_FEAT), lambda i: (i, 0)),
            pl.BlockSpec((NC, _TC_R, AUGD), lambda i: (0, i, 0)),
            pl.BlockSpec((IN_DIM, H2), fixed),
            pl.BlockSpec((1, H2), fixed),
            pl.BlockSpec((H2, EMB), fixed),
            pl.BlockSpec((1, EMB), fixed),
            pl.BlockSpec((EMB, H2), fixed),
            pl.BlockSpec((1, H2), fixed),
            pl.BlockSpec((H2, IN_DIM), fixed),
            pl.BlockSpec((1, IN_DIM), fixed),
        ],
        out_specs=[
            pl.BlockSpec((_TC_R, EMB), lambda i: (i, 0)),
            pl.BlockSpec((_TC_R, IN_DIM), lambda i: (i, 0)),
        ],
        out_shape=[
            jax.ShapeDtypeStruct((N_NODES, EMB), jnp.float32),
            jax.ShapeDtypeStruct((N_NODES, IN_DIM), jnp.float32),
        ],
    )(xp, parts, W_enc1, b_enc1.reshape(1, H2), W_enc3, b_enc3.reshape(1, EMB),
      W_dec1, b_dec1.reshape(1, H2), W_dec3, b_dec3.reshape(1, IN_DIM))
    return enc, dec


def kernel(x, edge_index, W_enc1, b_enc1, W_enc3, b_enc3,
           W_dec1, b_dec1, W_dec3, b_dec3):
    xaug = jnp.concatenate(
        [x, jnp.ones((N_NODES, 1), jnp.float32),
         jnp.zeros((N_NODES, AUGD - D_FEAT - 1), jnp.float32)], axis=1)
    src = edge_index[0]
    dst = edge_index[1]
    pad = NEP - N_EDGES
    srcp = jnp.concatenate([src, jnp.zeros((pad,), jnp.int32)]).reshape(NW * NCHUNK, K)
    pad_dst = N_NODES + jnp.arange(pad, dtype=jnp.int32) % (ROWS - N_NODES)
    dstp = jnp.concatenate([dst, pad_dst]).reshape(NW * NCHUNK, K)

    parts = _make_sc_agg()(xaug, srcp, dstp)

    enc, dec = _tc_dense(x, parts, W_enc1, b_enc1, W_enc3, b_enc3,
                         W_dec1, b_dec1, W_dec3, b_dec3)
    return enc, dec
